# baseline (device time: 37757 ns/iter reference)
import jax
import jax.numpy as jnp
from jax import lax
from jax.experimental import pallas as pl
from jax.experimental.pallas import tpu as pltpu

NX, NY, NZ = 2, 4, 4
NB = NX * NZ
NQH = 128
D = 128


def _collective_body(chunk_ref, out_ref, comm_a, comm_b,
                     send_a, recv_a, send_b, recv_b):
    my_x = lax.axis_index("x")
    my_y = lax.axis_index("y")
    my_z = lax.axis_index("z")
    bidx = my_x * NZ + my_z

    comm_a[my_y] = chunk_ref[:, :]

    a_sends = []
    for o in range(1, NY):
        ty = (my_y + o) % NY
        rdma = pltpu.make_async_remote_copy(
            src_ref=comm_a.at[my_y],
            dst_ref=comm_a.at[my_y],
            send_sem=send_a.at[o - 1],
            recv_sem=recv_a.at[o - 1],
            device_id=(my_x, ty, my_z),
            device_id_type=pl.DeviceIdType.MESH,
        )
        rdma.start()
        a_sends.append(rdma)

    for o in range(1, NY):
        j = (my_y - o + NY) % NY
        recv = pltpu.make_async_remote_copy(
            src_ref=comm_a.at[j],
            dst_ref=comm_a.at[j],
            send_sem=send_a.at[o - 1],
            recv_sem=recv_a.at[o - 1],
            device_id=(my_x, my_y, my_z),
            device_id_type=pl.DeviceIdType.MESH,
        )
        recv.wait_recv()

    total = comm_a[0] + comm_a[1] + comm_a[2] + comm_a[3]
    norm = total[0:NQH, :] / total[NQH:2 * NQH, :]
    comm_b[bidx] = norm

    b_sends = []
    for o in range(1, NB):
        tq = (bidx + o) % NB
        tx = tq // NZ
        tz = tq % NZ
        rdma = pltpu.make_async_remote_copy(
            src_ref=comm_b.at[bidx],
            dst_ref=comm_b.at[bidx],
            send_sem=send_b.at[o - 1],
            recv_sem=recv_b.at[o - 1],
            device_id=(tx, my_y, tz),
            device_id_type=pl.DeviceIdType.MESH,
        )
        rdma.start()
        b_sends.append(rdma)

    for o in range(1, NB):
        j = (bidx - o + NB) % NB
        recv = pltpu.make_async_remote_copy(
            src_ref=comm_b.at[j],
            dst_ref=comm_b.at[j],
            send_sem=send_b.at[o - 1],
            recv_sem=recv_b.at[o - 1],
            device_id=(my_x, my_y, my_z),
            device_id_type=pl.DeviceIdType.MESH,
        )
        recv.wait_recv()

    for s in range(NB):
        out_ref[s] = comm_b[s]

    for r in a_sends + b_sends:
        r.wait_send()


def _collective(chunk):
    return pl.pallas_call(
        _collective_body,
        out_shape=jax.ShapeDtypeStruct((NB, NQH, D), jnp.float32),
        in_specs=[pl.BlockSpec(memory_space=pltpu.VMEM)],
        out_specs=pl.BlockSpec(memory_space=pltpu.VMEM),
        scratch_shapes=[
            pltpu.VMEM((NY, 2 * NQH, D), jnp.float32),
            pltpu.VMEM((NB, NQH, D), jnp.float32),
            pltpu.SemaphoreType.DMA((NY - 1,)),
            pltpu.SemaphoreType.DMA((NY - 1,)),
            pltpu.SemaphoreType.DMA((NB - 1,)),
            pltpu.SemaphoreType.DMA((NB - 1,)),
        ],
    )(chunk)


def kernel(Q, K, V):
    my_x = lax.axis_index("x")
    my_z = lax.axis_index("z")
    bidx = my_x * NZ + my_z

    Qb = lax.dynamic_index_in_dim(Q, bidx, axis=0, keepdims=False)
    Kb = lax.dynamic_index_in_dim(K, bidx, axis=0, keepdims=False)
    Vb = lax.dynamic_index_in_dim(V, bidx, axis=0, keepdims=False)

    scale = 128 ** -0.5
    S = jnp.einsum("qhd,khd->qhk", Qb, Kb) * scale
    P = jnp.exp(S)
    l = jnp.sum(P, axis=-1)
    Op = jnp.einsum("qhk,khd->qhd", P, Vb)

    chunk = jnp.concatenate(
        [
            Op.reshape(NQH, D),
            jnp.broadcast_to(l.reshape(NQH)[:, None], (NQH, D)),
        ],
        axis=0,
    )

    out = _collective(chunk)
    return out.reshape(8, 8, 16, 128)


# device time: 21959 ns/iter; 1.7194x vs baseline; 1.7194x over previous
import jax
import jax.numpy as jnp
from jax import lax
from jax.experimental import pallas as pl
from jax.experimental.pallas import tpu as pltpu

NX, NY, NZ = 2, 4, 4
NB = NX * NZ
NQ = 8
NH = 16
NQH = NQ * NH
D = 128
KV = 1024
SCALE = D ** -0.5


def _body(Q_ref, K_ref, V_ref, out_ref,
          kv_k, kv_v, op_s, lb_s, comm_a, comm_b,
          k_sems, v_sems, send_a, recv_a, send_b, recv_b):
    my_x = lax.axis_index("x")
    my_y = lax.axis_index("y")
    my_z = lax.axis_index("z")
    bidx = my_x * NZ + my_z

    kdmas, vdmas = [], []
    for h in range(NH):
        kd = pltpu.make_async_copy(
            K_ref.at[bidx, :, h, :], kv_k.at[h], k_sems.at[h])
        vd = pltpu.make_async_copy(
            V_ref.at[bidx, :, h, :], kv_v.at[h], v_sems.at[h])
        kd.start()
        vd.start()
        kdmas.append(kd)
        vdmas.append(vd)

    barrier_sem = pltpu.get_barrier_semaphore()
    for o in range(1, NY):
        pl.semaphore_signal(
            barrier_sem, inc=1,
            device_id=(my_x, (my_y + o) % NY, my_z),
            device_id_type=pl.DeviceIdType.MESH,
        )
    for o in range(1, NB):
        tq = (bidx + o) % NB
        pl.semaphore_signal(
            barrier_sem, inc=1,
            device_id=(tq // NZ, my_y, tq % NZ),
            device_id_type=pl.DeviceIdType.MESH,
        )
    pl.semaphore_wait(barrier_sem, (NY - 1) + (NB - 1))

    Qb = Q_ref[bidx]
    for h in range(NH):
        kdmas[h].wait()
        vdmas[h].wait()
        qh = Qb[:, h, :]
        kh = kv_k[h]
        s = lax.dot_general(
            qh, kh, (((1,), (1,)), ((), ())),
            preferred_element_type=jnp.float32,
        )
        p = jnp.exp(s * SCALE)
        lh = jnp.sum(p, axis=1, keepdims=True)
        vh = kv_v[h]
        oh = lax.dot_general(
            p, vh, (((1,), (0,)), ((), ())),
            preferred_element_type=jnp.float32,
        )
        op_s[:, h, :] = oh
        lb_s[:, h, :] = jnp.broadcast_to(lh, (NQ, D))

    comm_a[my_y, 0:NQH, :] = op_s[...].reshape(NQH, D).astype(jnp.bfloat16)
    comm_a[my_y, NQH:2 * NQH, :] = lb_s[...].reshape(NQH, D).astype(jnp.bfloat16)

    a_sends = []
    for o in range(1, NY):
        ty = (my_y + o) % NY
        rdma = pltpu.make_async_remote_copy(
            src_ref=comm_a.at[my_y],
            dst_ref=comm_a.at[my_y],
            send_sem=send_a.at[o - 1],
            recv_sem=recv_a.at[o - 1],
            device_id=(my_x, ty, my_z),
            device_id_type=pl.DeviceIdType.MESH,
        )
        rdma.start()
        a_sends.append(rdma)

    for o in range(1, NY):
        j = (my_y - o + NY) % NY
        recv = pltpu.make_async_remote_copy(
            src_ref=comm_a.at[j],
            dst_ref=comm_a.at[j],
            send_sem=send_a.at[o - 1],
            recv_sem=recv_a.at[o - 1],
            device_id=(my_x, my_y, my_z),
            device_id_type=pl.DeviceIdType.MESH,
        )
        recv.wait_recv()

    total = (comm_a[0].astype(jnp.float32) + comm_a[1].astype(jnp.float32)
             + comm_a[2].astype(jnp.float32) + comm_a[3].astype(jnp.float32))
    norm = total[0:NQH, :] / total[NQH:2 * NQH, :]
    comm_b[bidx] = norm.astype(jnp.bfloat16)

    b_sends = []
    for o in range(1, NB):
        tq = (bidx + o) % NB
        rdma = pltpu.make_async_remote_copy(
            src_ref=comm_b.at[bidx],
            dst_ref=comm_b.at[bidx],
            send_sem=send_b.at[o - 1],
            recv_sem=recv_b.at[o - 1],
            device_id=(tq // NZ, my_y, tq % NZ),
            device_id_type=pl.DeviceIdType.MESH,
        )
        rdma.start()
        b_sends.append(rdma)

    for o in range(1, NB):
        j = (bidx - o + NB) % NB
        recv = pltpu.make_async_remote_copy(
            src_ref=comm_b.at[j],
            dst_ref=comm_b.at[j],
            send_sem=send_b.at[o - 1],
            recv_sem=recv_b.at[o - 1],
            device_id=(my_x, my_y, my_z),
            device_id_type=pl.DeviceIdType.MESH,
        )
        recv.wait_recv()

    for s in range(NB):
        out_ref[s] = comm_b[s].astype(jnp.float32)

    for r in a_sends + b_sends:
        r.wait_send()


def kernel(Q, K, V):
    out = pl.pallas_call(
        _body,
        out_shape=jax.ShapeDtypeStruct((NB, NQH, D), jnp.float32),
        in_specs=[
            pl.BlockSpec(memory_space=pltpu.VMEM),
            pl.BlockSpec(memory_space=pl.ANY),
            pl.BlockSpec(memory_space=pl.ANY),
        ],
        out_specs=pl.BlockSpec(memory_space=pltpu.VMEM),
        scratch_shapes=[
            pltpu.VMEM((NH, KV, D), jnp.float32),
            pltpu.VMEM((NH, KV, D), jnp.float32),
            pltpu.VMEM((NQ, NH, D), jnp.float32),
            pltpu.VMEM((NQ, NH, D), jnp.float32),
            pltpu.VMEM((NY, 2 * NQH, D), jnp.bfloat16),
            pltpu.VMEM((NB, NQH, D), jnp.bfloat16),
            pltpu.SemaphoreType.DMA((NH,)),
            pltpu.SemaphoreType.DMA((NH,)),
            pltpu.SemaphoreType.DMA((NY - 1,)),
            pltpu.SemaphoreType.DMA((NY - 1,)),
            pltpu.SemaphoreType.DMA((NB - 1,)),
            pltpu.SemaphoreType.DMA((NB - 1,)),
        ],
        compiler_params=pltpu.CompilerParams(collective_id=0),
    )(Q, K, V)
    return out.reshape(8, 8, 16, 128)
